# baseline (speedup 1.0000x reference)
import functools

import jax
import jax.numpy as jnp
from jax import lax
from jax.experimental import pallas as pl
from jax.experimental.pallas import tpu as pltpu
from jax.experimental.pallas import tpu_sc as plsc

_NW = 32
_L = 16
_BIG = 2**30


def _direction(b, qbase, n_q, m_t, src_q, src_t, out_d, out_i,
               tq, tt, od, oi, rbf, rbi):
    qx_h, qy_h, qz_h = src_q
    tx_h, ty_h, tz_h = src_t
    tqx, tqy, tqz = tq
    ttx, tty, ttz = tt
    pltpu.sync_copy(qx_h.at[b, pl.ds(qbase, n_q)], tqx)
    pltpu.sync_copy(qy_h.at[b, pl.ds(qbase, n_q)], tqy)
    pltpu.sync_copy(qz_h.at[b, pl.ds(qbase, n_q)], tqz)
    pltpu.sync_copy(tx_h.at[b], ttx)
    pltpu.sync_copy(ty_h.at[b], tty)
    pltpu.sync_copy(tz_h.at[b], ttz)

    iota16 = lax.iota(jnp.int32, _L)

    def _lane_min16(vec, buf, pad):
        buf[pl.ds(_L, _L)] = pad
        buf[pl.ds(0, _L)] = vec
        for sh in (8, 4, 2, 1):
            v = jnp.minimum(buf[pl.ds(0, _L)], buf[pl.ds(sh, _L)])
            buf[pl.ds(0, _L)] = v
        return buf[pl.ds(0, _L)][0]

    inf16 = jnp.full((_L,), jnp.inf, jnp.float32)
    big16 = jnp.full((_L,), _BIG, jnp.int32)

    def gbody(g, _):
        gb = g * _L
        qxv = tqx[pl.ds(gb, _L)]
        qyv = tqy[pl.ds(gb, _L)]
        qzv = tqz[pl.ds(gb, _L)]
        accd = jnp.zeros((_L,), jnp.float32)
        acci = jnp.zeros((_L,), jnp.int32)
        for k in range(_L):
            qx = qxv[k]
            qy = qyv[k]
            qz = qzv[k]

            def jbody(j, carry):
                best, bidx = carry
                base = j * _L
                dx = ttx[pl.ds(base, _L)] - qx
                dy = tty[pl.ds(base, _L)] - qy
                dz = ttz[pl.ds(base, _L)] - qz
                d = (dx * dx + dy * dy) + dz * dz
                m = d < best
                best = jnp.where(m, d, best)
                bidx = jnp.where(m, iota16 + base, bidx)
                return best, bidx

            best, bidx = lax.fori_loop(
                0, m_t // _L, jbody,
                (jnp.full((_L,), jnp.inf, jnp.float32),
                 jnp.zeros((_L,), jnp.int32)))
            mv = _lane_min16(best, rbf, inf16)
            ci = jnp.where(best == mv, bidx, _BIG)
            mi = _lane_min16(ci, rbi, big16)
            mk = iota16 == k
            accd = jnp.where(mk, mv, accd)
            acci = jnp.where(mk, mi, acci)
        od[pl.ds(gb, _L)] = accd
        oi[pl.ds(gb, _L)] = acci
        return 0

    lax.fori_loop(0, n_q // _L, gbody, 0)
    pltpu.sync_copy(od, out_d.at[b, pl.ds(qbase, n_q)])
    pltpu.sync_copy(oi, out_i.at[b, pl.ds(qbase, n_q)])


def kernel(xyz1, xyz2):
    B, N, _ = xyz1.shape
    M = xyz2.shape[1]
    n_q = N // _NW
    mesh = plsc.VectorSubcoreMesh(core_axis_name="c", subcore_axis_name="s")

    @functools.partial(
        pl.kernel, mesh=mesh,
        out_type=[
            jax.ShapeDtypeStruct((B, N), jnp.float32),
            jax.ShapeDtypeStruct((B, N), jnp.int32),
            jax.ShapeDtypeStruct((B, M), jnp.float32),
            jax.ShapeDtypeStruct((B, M), jnp.int32),
        ],
        scratch_types=[
            pltpu.VMEM((n_q,), jnp.float32),
            pltpu.VMEM((n_q,), jnp.float32),
            pltpu.VMEM((n_q,), jnp.float32),
            pltpu.VMEM((N,), jnp.float32),
            pltpu.VMEM((N,), jnp.float32),
            pltpu.VMEM((N,), jnp.float32),
            pltpu.VMEM((n_q,), jnp.float32),
            pltpu.VMEM((n_q,), jnp.int32),
            pltpu.VMEM((2 * _L,), jnp.float32),
            pltpu.VMEM((2 * _L,), jnp.int32),
        ],
    )
    def sc_chamfer(x1x, x1y, x1z, x2x, x2y, x2z,
                   d1_o, i1_o, d2_o, i2_o,
                   tqx, tqy, tqz, ttx, tty, ttz, od, oi, rbf, rbi):
        wid = lax.axis_index("s") * 2 + lax.axis_index("c")
        qbase = wid * n_q
        tq = (tqx, tqy, tqz)
        tt = (ttx, tty, ttz)
        def bbody(b, _):
            _direction(b, qbase, n_q, M, (x1x, x1y, x1z), (x2x, x2y, x2z),
                       d1_o, i1_o, tq, tt, od, oi, rbf, rbi)
            _direction(b, qbase, n_q, N, (x2x, x2y, x2z), (x1x, x1y, x1z),
                       d2_o, i2_o, tq, tt, od, oi, rbf, rbi)
            return 0

        lax.fori_loop(0, B, bbody, 0)

    x1p = jnp.transpose(xyz1, (2, 0, 1))
    x2p = jnp.transpose(xyz2, (2, 0, 1))
    dist1, idx1, dist2, idx2 = sc_chamfer(
        x1p[0], x1p[1], x1p[2], x2p[0], x2p[1], x2p[2])
    return dist1, dist2, idx1, idx2


# speedup vs baseline: 1.3905x; 1.3905x over previous
import functools

import jax
import jax.numpy as jnp
from jax import lax
from jax.experimental import pallas as pl
from jax.experimental.pallas import tpu as pltpu
from jax.experimental.pallas import tpu_sc as plsc

_NW = 32
_L = 16
_BIG = 2**30


def _direction(b, qbase, n_q, m_t, src_q, src_t, out_d, out_i,
               tq, tt, od, oi, rbf, rbi):
    qx_h, qy_h, qz_h = src_q
    tx_h, ty_h, tz_h = src_t
    tqx, tqy, tqz = tq
    ttx, tty, ttz = tt
    pltpu.sync_copy(qx_h.at[b, pl.ds(qbase, n_q)], tqx)
    pltpu.sync_copy(qy_h.at[b, pl.ds(qbase, n_q)], tqy)
    pltpu.sync_copy(qz_h.at[b, pl.ds(qbase, n_q)], tqz)
    pltpu.sync_copy(tx_h.at[b], ttx)
    pltpu.sync_copy(ty_h.at[b], tty)
    pltpu.sync_copy(tz_h.at[b], ttz)

    iota16 = lax.iota(jnp.int32, _L)

    def _lane_min16(vec, buf, pad):
        buf[pl.ds(_L, _L)] = pad
        buf[pl.ds(0, _L)] = vec
        for sh in (8, 4, 2, 1):
            v = jnp.minimum(buf[pl.ds(0, _L)], buf[pl.ds(sh, _L)])
            buf[pl.ds(0, _L)] = v
        return buf[pl.ds(0, _L)][0]

    inf16 = jnp.full((_L,), jnp.inf, jnp.float32)
    big16 = jnp.full((_L,), _BIG, jnp.int32)

    def gbody(g, _):
        gb = g * _L
        qxv = tqx[pl.ds(gb, _L)]
        qyv = tqy[pl.ds(gb, _L)]
        qzv = tqz[pl.ds(gb, _L)]
        accd = jnp.zeros((_L,), jnp.float32)
        acci = jnp.zeros((_L,), jnp.int32)
        for k in range(_L):
            qx = qxv[k]
            qy = qyv[k]
            qz = qzv[k]

            def jbody(j, carry):
                best, bidx = carry
                base = j * _L
                dx = ttx[pl.ds(base, _L)] - qx
                dy = tty[pl.ds(base, _L)] - qy
                dz = ttz[pl.ds(base, _L)] - qz
                d = (dx * dx + dy * dy) + dz * dz
                m = d < best
                best = jnp.where(m, d, best)
                bidx = jnp.where(m, iota16 + base, bidx)
                return best, bidx

            best, bidx = lax.fori_loop(
                0, m_t // _L, jbody,
                (jnp.full((_L,), jnp.inf, jnp.float32),
                 jnp.zeros((_L,), jnp.int32)),
                unroll=8)
            mv = _lane_min16(best, rbf, inf16)
            ci = jnp.where(best == mv, bidx, _BIG)
            mi = _lane_min16(ci, rbi, big16)
            mk = iota16 == k
            accd = jnp.where(mk, mv, accd)
            acci = jnp.where(mk, mi, acci)
        od[pl.ds(gb, _L)] = accd
        oi[pl.ds(gb, _L)] = acci
        return 0

    lax.fori_loop(0, n_q // _L, gbody, 0)
    pltpu.sync_copy(od, out_d.at[b, pl.ds(qbase, n_q)])
    pltpu.sync_copy(oi, out_i.at[b, pl.ds(qbase, n_q)])


def kernel(xyz1, xyz2):
    B, N, _ = xyz1.shape
    M = xyz2.shape[1]
    n_q = N // _NW
    mesh = plsc.VectorSubcoreMesh(core_axis_name="c", subcore_axis_name="s")

    @functools.partial(
        pl.kernel, mesh=mesh,
        out_type=[
            jax.ShapeDtypeStruct((B, N), jnp.float32),
            jax.ShapeDtypeStruct((B, N), jnp.int32),
            jax.ShapeDtypeStruct((B, M), jnp.float32),
            jax.ShapeDtypeStruct((B, M), jnp.int32),
        ],
        scratch_types=[
            pltpu.VMEM((n_q,), jnp.float32),
            pltpu.VMEM((n_q,), jnp.float32),
            pltpu.VMEM((n_q,), jnp.float32),
            pltpu.VMEM((N,), jnp.float32),
            pltpu.VMEM((N,), jnp.float32),
            pltpu.VMEM((N,), jnp.float32),
            pltpu.VMEM((n_q,), jnp.float32),
            pltpu.VMEM((n_q,), jnp.int32),
            pltpu.VMEM((2 * _L,), jnp.float32),
            pltpu.VMEM((2 * _L,), jnp.int32),
        ],
    )
    def sc_chamfer(x1x, x1y, x1z, x2x, x2y, x2z,
                   d1_o, i1_o, d2_o, i2_o,
                   tqx, tqy, tqz, ttx, tty, ttz, od, oi, rbf, rbi):
        wid = lax.axis_index("s") * 2 + lax.axis_index("c")
        qbase = wid * n_q
        tq = (tqx, tqy, tqz)
        tt = (ttx, tty, ttz)
        def bbody(b, _):
            _direction(b, qbase, n_q, M, (x1x, x1y, x1z), (x2x, x2y, x2z),
                       d1_o, i1_o, tq, tt, od, oi, rbf, rbi)
            _direction(b, qbase, n_q, N, (x2x, x2y, x2z), (x1x, x1y, x1z),
                       d2_o, i2_o, tq, tt, od, oi, rbf, rbi)
            return 0

        lax.fori_loop(0, B, bbody, 0)

    x1p = jnp.transpose(xyz1, (2, 0, 1))
    x2p = jnp.transpose(xyz2, (2, 0, 1))
    dist1, idx1, dist2, idx2 = sc_chamfer(
        x1p[0], x1p[1], x1p[2], x2p[0], x2p[1], x2p[2])
    return dist1, dist2, idx1, idx2


# speedup vs baseline: 6.5727x; 4.7269x over previous
import functools

import jax
import jax.numpy as jnp
from jax import lax
from jax.experimental import pallas as pl
from jax.experimental.pallas import tpu as pltpu
from jax.experimental.pallas import tpu_sc as plsc

_TM = 512
_RC = 32
_BIG = 2**30
_NW = 32
_L = 16
_SCB = 1


def _tc_body(x1_ref, x2t_ref, dist1_ref, idx1_ref, dist2_ref, idx2_ref,
             qv_s, qi_s):
    mt = pl.program_id(1)
    nmt = pl.num_programs(1)
    n = x1_ref.shape[1]
    x2t = x2t_ref[0]
    x2x = x2t[0:1, :]
    x2y = x2t[1:2, :]
    x2z = x2t[2:3, :]
    lane = lax.broadcasted_iota(jnp.int32, (_RC, 128), 1)
    ibase = [lane + (t * 128 + mt * _TM) for t in range(_TM // 128)]

    @pl.when(mt == 0)
    def _():
        qv_s[...] = jnp.full((n, 128), jnp.inf, jnp.float32)

    pv = None
    pa = None
    for c in range(n // _RC):
        r0 = c * _RC
        x1c = x1_ref[0, r0:r0 + _RC, :]
        dx = x1c[:, 0:1] - x2x
        dy = x1c[:, 1:2] - x2y
        dz = x1c[:, 2:3] - x2z
        d = (dx * dx + dy * dy) + dz * dz

        d3 = d.reshape(_RC // 8, 8, _TM)
        for a in range(_RC // 8):
            da = d3[a]
            ag = c * (_RC // 8) + a
            if pv is None:
                pv = da
                pa = jnp.zeros((8, _TM), jnp.int32)
            else:
                m = da < pv
                pv = jnp.where(m, da, pv)
                pa = jnp.where(m, ag, pa)

        qv = d[:, 0:128]
        qi = ibase[0]
        for t in range(1, _TM // 128):
            dt = d[:, t * 128:(t + 1) * 128]
            m = dt < qv
            qv = jnp.where(m, dt, qv)
            qi = jnp.where(m, ibase[t], qi)

        prev = qv_s[r0:r0 + _RC, :]
        m2 = qv < prev
        qv_s[r0:r0 + _RC, :] = jnp.where(m2, qv, prev)
        qi_s[r0:r0 + _RC, :] = jnp.where(m2, qi, qi_s[r0:r0 + _RC, :])

    rowidx = pa * 8 + lax.broadcasted_iota(jnp.int32, (8, _TM), 0)
    fv = jnp.min(pv, axis=0)
    fi = jnp.min(jnp.where(pv == fv[None, :], rowidx, _BIG), axis=0)
    dist2_ref[0, 0, :] = fv
    idx2_ref[0, 0, :] = fi

    @pl.when(mt == nmt - 1)
    def _():
        qvf = qv_s[...]
        qif = qi_s[...]
        gv = jnp.min(qvf, axis=1, keepdims=True)
        gi = jnp.min(jnp.where(qvf == gv, qif, _BIG), axis=1, keepdims=True)
        dist1_ref[0, :, :] = gv
        idx1_ref[0, :, :] = gi


def _tc_chamfer(xyz1, xyz2):
    B, N, _ = xyz1.shape
    M = xyz2.shape[1]
    x2t = jnp.transpose(xyz2, (0, 2, 1))
    grid = (B, M // _TM)

    dist1, idx1, dist2, idx2 = pl.pallas_call(
        _tc_body,
        grid=grid,
        in_specs=[
            pl.BlockSpec((1, N, 3), lambda b, mt: (b, 0, 0)),
            pl.BlockSpec((1, 3, _TM), lambda b, mt: (b, 0, mt)),
        ],
        out_specs=[
            pl.BlockSpec((1, N, 1), lambda b, mt: (b, 0, 0)),
            pl.BlockSpec((1, N, 1), lambda b, mt: (b, 0, 0)),
            pl.BlockSpec((1, 1, _TM), lambda b, mt: (b, 0, mt)),
            pl.BlockSpec((1, 1, _TM), lambda b, mt: (b, 0, mt)),
        ],
        out_shape=[
            jax.ShapeDtypeStruct((B, N, 1), jnp.float32),
            jax.ShapeDtypeStruct((B, N, 1), jnp.int32),
            jax.ShapeDtypeStruct((B, 1, M), jnp.float32),
            jax.ShapeDtypeStruct((B, 1, M), jnp.int32),
        ],
        scratch_shapes=[
            pltpu.VMEM((N, 128), jnp.float32),
            pltpu.VMEM((N, 128), jnp.int32),
        ],
        compiler_params=pltpu.CompilerParams(
            dimension_semantics=("parallel", "arbitrary"),
        ),
    )(xyz1, x2t)

    return (
        dist1.reshape(B, N),
        dist2.reshape(B, M),
        idx1.reshape(B, N),
        idx2.reshape(B, M),
    )


def _direction(b, qbase, n_q, m_t, src_q, src_t, out_d, out_i,
               tq, tt, od, oi, rbf, rbi):
    qx_h, qy_h, qz_h = src_q
    tx_h, ty_h, tz_h = src_t
    tqx, tqy, tqz = tq
    ttx, tty, ttz = tt
    pltpu.sync_copy(qx_h.at[b, pl.ds(qbase, n_q)], tqx)
    pltpu.sync_copy(qy_h.at[b, pl.ds(qbase, n_q)], tqy)
    pltpu.sync_copy(qz_h.at[b, pl.ds(qbase, n_q)], tqz)
    pltpu.sync_copy(tx_h.at[b], ttx)
    pltpu.sync_copy(ty_h.at[b], tty)
    pltpu.sync_copy(tz_h.at[b], ttz)

    iota16 = lax.iota(jnp.int32, _L)

    def _lane_min16(vec, buf, pad):
        buf[pl.ds(_L, _L)] = pad
        buf[pl.ds(0, _L)] = vec
        for sh in (8, 4, 2, 1):
            v = jnp.minimum(buf[pl.ds(0, _L)], buf[pl.ds(sh, _L)])
            buf[pl.ds(0, _L)] = v
        return buf[pl.ds(0, _L)][0]

    inf16 = jnp.full((_L,), jnp.inf, jnp.float32)
    big16 = jnp.full((_L,), _BIG, jnp.int32)

    def gbody(g, _):
        gb = g * _L
        qxv = tqx[pl.ds(gb, _L)]
        qyv = tqy[pl.ds(gb, _L)]
        qzv = tqz[pl.ds(gb, _L)]
        accd = jnp.zeros((_L,), jnp.float32)
        acci = jnp.zeros((_L,), jnp.int32)
        for k in range(_L):
            qx = qxv[k]
            qy = qyv[k]
            qz = qzv[k]

            def jbody(j, carry):
                best, bidx = carry
                base = j * _L
                dx = ttx[pl.ds(base, _L)] - qx
                dy = tty[pl.ds(base, _L)] - qy
                dz = ttz[pl.ds(base, _L)] - qz
                d = (dx * dx + dy * dy) + dz * dz
                m = d < best
                best = jnp.where(m, d, best)
                bidx = jnp.where(m, iota16 + base, bidx)
                return best, bidx

            best, bidx = lax.fori_loop(
                0, m_t // _L, jbody,
                (jnp.full((_L,), jnp.inf, jnp.float32),
                 jnp.zeros((_L,), jnp.int32)),
                unroll=8)
            mv = _lane_min16(best, rbf, inf16)
            ci = jnp.where(best == mv, bidx, _BIG)
            mi = _lane_min16(ci, rbi, big16)
            mk = iota16 == k
            accd = jnp.where(mk, mv, accd)
            acci = jnp.where(mk, mi, acci)
        od[pl.ds(gb, _L)] = accd
        oi[pl.ds(gb, _L)] = acci
        return 0

    lax.fori_loop(0, n_q // _L, gbody, 0)
    pltpu.sync_copy(od, out_d.at[b, pl.ds(qbase, n_q)])
    pltpu.sync_copy(oi, out_i.at[b, pl.ds(qbase, n_q)])


def _sc_chamfer(xyz1, xyz2):
    B, N, _ = xyz1.shape
    M = xyz2.shape[1]
    n_q = N // _NW
    mesh = plsc.VectorSubcoreMesh(core_axis_name="c", subcore_axis_name="s")

    @functools.partial(
        pl.kernel, mesh=mesh,
        out_type=[
            jax.ShapeDtypeStruct((B, N), jnp.float32),
            jax.ShapeDtypeStruct((B, N), jnp.int32),
            jax.ShapeDtypeStruct((B, M), jnp.float32),
            jax.ShapeDtypeStruct((B, M), jnp.int32),
        ],
        scratch_types=[
            pltpu.VMEM((n_q,), jnp.float32),
            pltpu.VMEM((n_q,), jnp.float32),
            pltpu.VMEM((n_q,), jnp.float32),
            pltpu.VMEM((N,), jnp.float32),
            pltpu.VMEM((N,), jnp.float32),
            pltpu.VMEM((N,), jnp.float32),
            pltpu.VMEM((n_q,), jnp.float32),
            pltpu.VMEM((n_q,), jnp.int32),
            pltpu.VMEM((2 * _L,), jnp.float32),
            pltpu.VMEM((2 * _L,), jnp.int32),
        ],
    )
    def sc_kernel(x1x, x1y, x1z, x2x, x2y, x2z,
                  d1_o, i1_o, d2_o, i2_o,
                  tqx, tqy, tqz, ttx, tty, ttz, od, oi, rbf, rbi):
        wid = lax.axis_index("s") * 2 + lax.axis_index("c")
        qbase = wid * n_q
        tq = (tqx, tqy, tqz)
        tt = (ttx, tty, ttz)

        def bbody(b, _):
            _direction(b, qbase, n_q, M, (x1x, x1y, x1z), (x2x, x2y, x2z),
                       d1_o, i1_o, tq, tt, od, oi, rbf, rbi)
            _direction(b, qbase, n_q, N, (x2x, x2y, x2z), (x1x, x1y, x1z),
                       d2_o, i2_o, tq, tt, od, oi, rbf, rbi)
            return 0

        lax.fori_loop(0, B, bbody, 0)

    x1p = jnp.transpose(xyz1, (2, 0, 1))
    x2p = jnp.transpose(xyz2, (2, 0, 1))
    dist1, idx1, dist2, idx2 = sc_kernel(
        x1p[0], x1p[1], x1p[2], x2p[0], x2p[1], x2p[2])
    return dist1, dist2, idx1, idx2


def kernel(xyz1, xyz2):
    sc_out = _sc_chamfer(xyz1[:_SCB], xyz2[:_SCB])
    tc_out = _tc_chamfer(xyz1[_SCB:], xyz2[_SCB:])
    return tuple(jnp.concatenate([s, t], axis=0)
                 for s, t in zip(sc_out, tc_out))
